# baseline (speedup 1.0000x reference)
import functools

import jax
import jax.numpy as jnp
from jax.experimental import pallas as pl
from jax.experimental.pallas import tpu as pltpu

_S = 7


def _roi_align_body(fm_hbm, p_ref, out_ref, fm_vmem, sem, *, n_boxes,
                    rc_size, h, w):
    b = pl.program_id(0)
    rc = pl.program_id(1)

    @pl.when(rc == 0)
    def _():
        cp = pltpu.make_async_copy(fm_hbm.at[b], fm_vmem.at[:, pl.ds(0, w), :],
                                   sem)
        cp.start()
        cp.wait()

    f32 = jnp.float32

    def roi_body(rr, carry):
      rb = pl.multiple_of(rr * 8, 8)
      for u in range(8):
        r = rb + u
        g = (b * n_boxes + rc * rc_size + r) * 4
        x1 = jnp.clip(p_ref[g + 0], 0, w - 1)
        y1 = jnp.clip(p_ref[g + 1], 0, h - 1)
        x2 = jnp.clip(p_ref[g + 2], 0, w - 1)
        y2 = jnp.clip(p_ref[g + 3], 0, h - 1)
        scale_y = (y2 - y1 + 1).astype(f32) / _S
        scale_x = (x2 - x1 + 1).astype(f32) / _S
        ys = []
        for i in range(_S):
            sy = jnp.maximum(f32(i + 0.5) * scale_y - f32(0.5), f32(0.0))
            y0f = jnp.floor(sy)
            wy = sy - y0f
            ya0 = y1 + y0f.astype(jnp.int32)
            ya1 = jnp.minimum(ya0 + 1, y2)
            ys.append((ya0, ya1, wy))
        xs = []
        for j in range(_S):
            sx = jnp.maximum(f32(j + 0.5) * scale_x - f32(0.5), f32(0.0))
            x0f = jnp.floor(sx)
            wx = sx - x0f
            xa0 = x1 + x0f.astype(jnp.int32)
            wx = jnp.where(xa0 >= x2, f32(0.0), wx)
            xb = pl.multiple_of((xa0 >> 3) << 3, 8)
            xs.append((xb, xa0 & 7, wx))
        for i in range(_S):
            ya0, ya1, wy = ys[i]
            omy = f32(1.0) - wy
            for j in range(_S):
                xb, k, wx = xs[j]
                omx = f32(1.0) - wx
                c0 = fm_vmem[ya0, pl.ds(xb, 16), :]
                c1 = fm_vmem[ya1, pl.ds(xb, 16), :]
                ty = omy * c0 + wy * c1
                tp = pltpu.roll(ty, -k, axis=0)
                acc = omx * tp[0] + wx * tp[1]
                out_ref[i * _S + j, r, :] = acc
      return carry

    jax.lax.fori_loop(0, rc_size // 8, roi_body, 0)


def kernel(feature_map, proposals):
    B, C, H, W = feature_map.shape
    N = proposals.shape[1]
    RC = 64
    while N % RC:
        RC //= 2
    fm = jnp.transpose(feature_map, (0, 2, 3, 1))
    props = proposals.reshape(-1).astype(jnp.int32)
    n_rc = N // RC
    body = functools.partial(_roi_align_body, n_boxes=N, rc_size=RC, h=H, w=W)
    out = pl.pallas_call(
        body,
        grid=(B, n_rc),
        in_specs=[pl.BlockSpec(memory_space=pl.ANY),
                  pl.BlockSpec(memory_space=pltpu.SMEM)],
        out_specs=pl.BlockSpec((_S * _S, RC, C),
                               lambda b, rc: (0, b * n_rc + rc, 0)),
        out_shape=jax.ShapeDtypeStruct((_S * _S, B * N, C), jnp.float32),
        scratch_shapes=[pltpu.VMEM((H, W + 16, C), jnp.float32),
                        pltpu.SemaphoreType.DMA],
        compiler_params=pltpu.CompilerParams(
            dimension_semantics=("parallel", "arbitrary"),
            vmem_limit_bytes=54 * 1024 * 1024,
        ),
        name="roi_align",
    )(fm, props)
    return out.transpose(1, 2, 0).reshape(B * N, C, _S, _S)


# speedup vs baseline: 1.0236x; 1.0236x over previous
import functools

import jax
import jax.numpy as jnp
from jax.experimental import pallas as pl
from jax.experimental.pallas import tpu as pltpu

_S = 7


def _roi_align_body(fm_hbm, p_ref, out_ref, fm_vmem, sem, *, n_boxes,
                    rc_size, h, w):
    b = pl.program_id(0)
    rc = pl.program_id(1)

    @pl.when(rc == 0)
    def _():
        cp = pltpu.make_async_copy(fm_hbm.at[b], fm_vmem.at[:, pl.ds(0, w), :],
                                   sem)
        cp.start()
        cp.wait()

    f32 = jnp.float32

    def roi_body(rr, carry):
      rb = pl.multiple_of(rr * 8, 8)
      for u in range(8):
        r = rb + u
        g = (b * n_boxes + rc * rc_size + r) * 4
        x1 = jnp.clip(p_ref[g + 0], 0, w - 1)
        y1 = jnp.clip(p_ref[g + 1], 0, h - 1)
        x2 = jnp.clip(p_ref[g + 2], 0, w - 1)
        y2 = jnp.clip(p_ref[g + 3], 0, h - 1)
        scale_y = (y2 - y1 + 1).astype(f32) / _S
        scale_x = (x2 - x1 + 1).astype(f32) / _S
        ys = []
        for i in range(_S):
            sy = jnp.maximum(f32(i + 0.5) * scale_y - f32(0.5), f32(0.0))
            y0f = jnp.floor(sy)
            wy = sy - y0f
            ya0 = y1 + y0f.astype(jnp.int32)
            ya1 = jnp.minimum(ya0 + 1, y2)
            ys.append((ya0, ya1, wy))
        xs = []
        for j in range(_S):
            sx = jnp.maximum(f32(j + 0.5) * scale_x - f32(0.5), f32(0.0))
            x0f = jnp.floor(sx)
            wx = sx - x0f
            xa0 = x1 + x0f.astype(jnp.int32)
            wx = jnp.where(xa0 >= x2, f32(0.0), wx)
            xb = pl.multiple_of((xa0 >> 3) << 3, 8)
            xs.append((xb, xa0 & 7, wx))
        for i in range(_S):
            ya0, ya1, wy = ys[i]
            omy = f32(1.0) - wy
            for j in range(_S):
                xb, k, wx = xs[j]
                omx = f32(1.0) - wx
                c0 = fm_vmem[ya0, pl.ds(xb, 16), :]
                c1 = fm_vmem[ya1, pl.ds(xb, 16), :]
                ty = omy * c0 + wy * c1
                tp = pltpu.roll(ty, -k, axis=0)
                acc = omx * tp[0] + wx * tp[1]
                out_ref[i * _S + j, r, :] = acc
      return carry

    jax.lax.fori_loop(0, rc_size // 8, roi_body, 0)


def kernel(feature_map, proposals):
    B, C, H, W = feature_map.shape
    N = proposals.shape[1]
    RC = 32
    while N % RC:
        RC //= 2
    fm = jnp.transpose(feature_map, (0, 2, 3, 1))
    props = proposals.reshape(-1).astype(jnp.int32)
    n_rc = N // RC
    body = functools.partial(_roi_align_body, n_boxes=N, rc_size=RC, h=H, w=W)
    out = pl.pallas_call(
        body,
        grid=(B, n_rc),
        in_specs=[pl.BlockSpec(memory_space=pl.ANY),
                  pl.BlockSpec(memory_space=pltpu.SMEM)],
        out_specs=pl.BlockSpec((_S * _S, RC, C),
                               lambda b, rc: (0, b * n_rc + rc, 0)),
        out_shape=jax.ShapeDtypeStruct((_S * _S, B * N, C), jnp.float32),
        scratch_shapes=[pltpu.VMEM((H, W + 16, C), jnp.float32),
                        pltpu.SemaphoreType.DMA],
        compiler_params=pltpu.CompilerParams(
            dimension_semantics=("parallel", "arbitrary"),
            vmem_limit_bytes=54 * 1024 * 1024,
        ),
        name="roi_align",
    )(fm, props)
    return out.transpose(1, 2, 0).reshape(B * N, C, _S, _S)
